# Initial kernel scaffold; baseline (speedup 1.0000x reference)
import jax
import jax.numpy as jnp
from jax.experimental import pallas as pl


def kernel(q, k, v, pos_bias):
    raise NotImplementedError("write your pallas kernel here")


# speedup vs baseline: 7.6845x; 7.6845x over previous
import jax
import jax.numpy as jnp
from jax.experimental import pallas as pl

_OFFSETS = tuple(list(range(33)) + [48, 64, 96, 128, 192, 256, 384, 512, 768, 1024, 1536])
_STRIDED = _OFFSETS[33:]
_NWIN = 33
_PAD = 1536
_NEG = -1e30
_TILE = 128
_KWIN = _TILE + _NWIN - 1


def _attn_body(w_ref, ws_ref, q_ref, kp_ref, vp_ref, out_ref):
    wband = w_ref[0]
    wstr = ws_ref[0]

    r2 = jax.lax.broadcasted_iota(jnp.int32, (_TILE, _KWIN), 0)
    c2 = jax.lax.broadcasted_iota(jnp.int32, (_TILE, _KWIN), 1)
    band = (c2 >= r2) & (c2 <= r2 + (_NWIN - 1))
    rcol = jax.lax.broadcasted_iota(jnp.int32, (_TILE, 1), 0)

    def tile(t, _):
        n0 = t * _TILE
        qb = q_ref[0, pl.ds(n0, _TILE), :]
        kw = kp_ref[0, pl.ds(n0 + _PAD - (_NWIN - 1), _KWIN), :]
        s_win = jax.lax.dot_general(
            qb, kw, (((1,), (1,)), ((), ())),
            preferred_element_type=jnp.float32)
        valid = band & ((c2 + n0 - (_NWIN - 1)) >= 0)
        s_win = jnp.where(valid, s_win, _NEG)
        m = jnp.max(s_win, axis=1, keepdims=True)

        s_str = []
        for off in _STRIDED:
            kb = kp_ref[0, pl.ds(n0 + _PAD - off, _TILE), :]
            sj = jnp.sum(qb * kb, axis=1, keepdims=True)
            sj = jnp.where(rcol + n0 >= off, sj, _NEG)
            s_str.append(sj)
        s_strm = jnp.concatenate(s_str, axis=1)
        m = jnp.maximum(m, jnp.max(s_strm, axis=1, keepdims=True))

        aw = jnp.exp(s_win - m) * wband
        vw = vp_ref[0, pl.ds(n0 + _PAD - (_NWIN - 1), _KWIN), :]
        num = jax.lax.dot_general(
            aw, vw, (((1,), (0,)), ((), ())),
            preferred_element_type=jnp.float32)

        esw = jnp.exp(s_strm - m) * wstr[:, :11]
        for j, off in enumerate(_STRIDED):
            vb = vp_ref[0, pl.ds(n0 + _PAD - off, _TILE), :]
            num = num + esw[:, j:j + 1] * vb
        den = num[:, 64:65]
        out_ref[0, pl.ds(n0, _TILE), :] = num[:, :64] / den
        return 0

    jax.lax.fori_loop(0, 2048 // _TILE, tile, 0, unroll=False)


def kernel(q, k, v, pos_bias):
    B, H, N, HD = q.shape
    sc = 1.0 / (HD ** 0.5)
    qs = q[0] * sc
    kp = jnp.pad(k[0], ((0, 0), (_PAD, 0), (0, 0)))
    vo = jnp.concatenate([v[0], jnp.ones((H, N, 1), v.dtype)], axis=2)
    vp = jnp.pad(vo, ((0, 0), (_PAD, 0), (0, 7)))

    r = jnp.arange(_TILE)[:, None]
    c = jnp.arange(_KWIN)[None, :]
    off_mat = r + (_NWIN - 1) - c
    band = (off_mat >= 0) & (off_mat < _NWIN)
    wb = jnp.exp(pos_bias[jnp.clip(off_mat, 0, _NWIN - 1)])
    wband = jnp.where(band[..., None], wb, 0.0).transpose(2, 0, 1)
    wstr = jnp.exp(pos_bias[_NWIN:, :]).T
    wstr = jnp.pad(wstr, ((0, 0), (0, 5)))[:, None, :]

    out = pl.pallas_call(
        _attn_body,
        grid=(H,),
        in_specs=[
            pl.BlockSpec((1, _TILE, _KWIN), lambda h: (h, 0, 0)),
            pl.BlockSpec((1, 1, 16), lambda h: (h, 0, 0)),
            pl.BlockSpec((1, N, HD), lambda h: (h, 0, 0)),
            pl.BlockSpec((1, N + _PAD, HD), lambda h: (h, 0, 0)),
            pl.BlockSpec((1, N + _PAD, 72), lambda h: (h, 0, 0)),
        ],
        out_specs=pl.BlockSpec((1, N, HD), lambda h: (h, 0, 0)),
        out_shape=jax.ShapeDtypeStruct((H, N, HD), jnp.float32),
    )(wband, wstr, qs, kp, vp)
    return out[None]


# speedup vs baseline: 10.8261x; 1.4088x over previous
import jax
import jax.numpy as jnp
from jax.experimental import pallas as pl

_OFFSETS = tuple(list(range(33)) + [48, 64, 96, 128, 192, 256, 384, 512, 768, 1024, 1536])
_LB = 128
_NCOV = 37
_STRIDED = tuple(o for o in _OFFSETS if o > _LB)
_PAD = 1536
_NEG = -1e30
_TILE = 128
_KWIN = _TILE + _LB
_SC = 0.125


def _attn_body(w_ref, ws_ref, q_ref, kp_ref, vp_ref, out_ref):
    wband = w_ref[0]
    wstr = ws_ref[0]
    wpos = wband > 0.0

    c2 = jax.lax.broadcasted_iota(jnp.int32, (_TILE, _KWIN), 1)
    rcol = jax.lax.broadcasted_iota(jnp.int32, (_TILE, 1), 0)

    def tile(t, _):
        n0 = pl.multiple_of(t * _TILE, _TILE)
        qb = q_ref[0, pl.ds(n0, _TILE), :] * _SC
        kw = kp_ref[0, pl.ds(n0 + _PAD - _LB, _KWIN), :]
        s_win = jax.lax.dot_general(
            qb, kw, (((1,), (1,)), ((), ())),
            preferred_element_type=jnp.float32)
        valid = wpos & ((c2 + n0 - _LB) >= 0)
        s_win = jnp.where(valid, s_win, _NEG)
        m = jnp.max(s_win, axis=1, keepdims=True)

        s_str = []
        for off in _STRIDED:
            kb = kp_ref[0, pl.ds(n0 + _PAD - off, _TILE), :]
            sj = jnp.sum(qb * kb, axis=1, keepdims=True)
            sj = jnp.where(rcol + n0 >= off, sj, _NEG)
            s_str.append(sj)
        s_strm = jnp.concatenate(s_str, axis=1)
        m = jnp.maximum(m, jnp.max(s_strm, axis=1, keepdims=True))

        aw = jnp.exp(s_win - m) * wband
        vw = vp_ref[0, pl.ds(n0 + _PAD - _LB, _KWIN), :]
        num = jax.lax.dot_general(
            aw, vw, (((1,), (0,)), ((), ())),
            preferred_element_type=jnp.float32)

        esw = jnp.exp(s_strm - m) * wstr[:, :7]
        for j, off in enumerate(_STRIDED):
            vb = vp_ref[0, pl.ds(n0 + _PAD - off, _TILE), :]
            num = num + esw[:, j:j + 1] * vb
        den = num[:, 64:65]
        out_ref[0, pl.ds(n0, _TILE), :] = num[:, :64] / den
        return 0

    jax.lax.fori_loop(0, 2048 // _TILE, tile, 0, unroll=False)


def kernel(q, k, v, pos_bias):
    B, H, N, HD = q.shape
    kp = jnp.pad(k[0], ((0, 0), (_PAD, 0), (0, 0)))
    vo = jnp.concatenate([v[0], jnp.ones((H, N, 1), v.dtype)], axis=2)
    vp = jnp.pad(vo, ((0, 0), (_PAD, 0), (0, 7)))

    cov = jnp.array([o for o in _OFFSETS if o <= _LB], jnp.int32)
    r = jnp.arange(_TILE)[:, None]
    c = jnp.arange(_KWIN)[None, :]
    off_mat = r + _LB - c
    onehot = (off_mat[:, :, None] == cov[None, None, :]).astype(jnp.float32)
    eb = jnp.exp(pos_bias[:_NCOV, :])
    wband = jnp.einsum('rci,ih->hrc', onehot, eb)
    wstr = jnp.exp(pos_bias[_NCOV:, :]).T
    wstr = jnp.pad(wstr, ((0, 0), (0, 1)))[:, None, :]

    out = pl.pallas_call(
        _attn_body,
        grid=(H,),
        in_specs=[
            pl.BlockSpec((1, _TILE, _KWIN), lambda h: (h, 0, 0)),
            pl.BlockSpec((1, 1, 8), lambda h: (h, 0, 0)),
            pl.BlockSpec((1, N, HD), lambda h: (h, 0, 0)),
            pl.BlockSpec((1, N + _PAD, HD), lambda h: (h, 0, 0)),
            pl.BlockSpec((1, N + _PAD, 72), lambda h: (h, 0, 0)),
        ],
        out_specs=pl.BlockSpec((1, N, HD), lambda h: (h, 0, 0)),
        out_shape=jax.ShapeDtypeStruct((H, N, HD), jnp.float32),
    )(wband, wstr, q[0], kp, vp)
    return out[None]


# speedup vs baseline: 23.8276x; 2.2010x over previous
import jax
import jax.numpy as jnp
from jax.experimental import pallas as pl

_OFFSETS = tuple(list(range(33)) + [48, 64, 96, 128, 192, 256, 384, 512, 768, 1024, 1536])
_LB = 128
_NCOV = 37
_STRIDED = tuple(o for o in _OFFSETS if o > _LB)
_NEG = -1e30
_TILE = 128
_KWIN = _TILE + _LB
_N = 2048
_SC = 0.125


def _attn_body(w_ref, ws_ref, q_ref, k_ref, v_ref, out_ref):
    wband = w_ref[0]
    wstr = ws_ref[0]
    wpos = wband > 0.0

    c2 = jax.lax.broadcasted_iota(jnp.int32, (_TILE, _KWIN), 1)
    rcol = jax.lax.broadcasted_iota(jnp.int32, (_TILE, 1), 0)

    for t in range(_N // _TILE):
        n0 = t * _TILE
        qb = q_ref[0, pl.ds(n0, _TILE), :] * _SC
        if t == 0:
            kw = jnp.concatenate(
                [k_ref[0, 0:_TILE, :], k_ref[0, 0:_TILE, :]], axis=0)
            vw = jnp.concatenate(
                [v_ref[0, 0:_TILE, :], v_ref[0, 0:_TILE, :]], axis=0)
        else:
            kw = k_ref[0, pl.ds(n0 - _LB, _KWIN), :]
            vw = v_ref[0, pl.ds(n0 - _LB, _KWIN), :]
        s_win = jax.lax.dot_general(
            qb, kw, (((1,), (1,)), ((), ())),
            preferred_element_type=jnp.float32)
        if t == 0:
            valid = wpos & (c2 >= _LB)
        else:
            valid = wpos
        s_win = jnp.where(valid, s_win, _NEG)
        m = jnp.max(s_win, axis=1, keepdims=True)

        live = [(j, off) for j, off in enumerate(_STRIDED) if n0 + _TILE > off]
        s_str = {}
        for j, off in live:
            if n0 >= off:
                kb = k_ref[0, pl.ds(n0 - off, _TILE), :]
            else:
                d = off - n0
                kb = jnp.concatenate(
                    [k_ref[0, 0:d, :], k_ref[0, 0:_TILE - d, :]], axis=0)
            sj = jnp.sum(qb * kb, axis=1, keepdims=True)
            if n0 < off:
                sj = jnp.where(rcol >= off - n0, sj, _NEG)
            s_str[j] = sj
            m = jnp.maximum(m, sj)

        aw = jnp.exp(s_win - m) * wband
        num = jax.lax.dot_general(
            aw, vw, (((1,), (0,)), ((), ())),
            preferred_element_type=jnp.float32)
        den = jnp.sum(aw, axis=1, keepdims=True)

        for j, off in live:
            ej = jnp.exp(s_str[j] - m) * wstr[0, j]
            if n0 >= off:
                vb = v_ref[0, pl.ds(n0 - off, _TILE), :]
            else:
                d = off - n0
                ej = jnp.where(rcol >= d, ej, 0.0)
                vb = jnp.concatenate(
                    [v_ref[0, 0:d, :], v_ref[0, 0:_TILE - d, :]], axis=0)
            num = num + ej * vb
            den = den + ej
        out_ref[0, pl.ds(n0, _TILE), :] = num / den


def kernel(q, k, v, pos_bias):
    B, H, N, HD = q.shape

    cov = jnp.array([o for o in _OFFSETS if o <= _LB], jnp.int32)
    r = jnp.arange(_TILE)[:, None]
    c = jnp.arange(_KWIN)[None, :]
    off_mat = r + _LB - c
    onehot = (off_mat[:, :, None] == cov[None, None, :]).astype(jnp.float32)
    eb = jnp.exp(pos_bias[:_NCOV, :])
    wband = jnp.einsum('rci,ih->hrc', onehot, eb)
    wstr = jnp.exp(pos_bias[_NCOV:, :]).T
    wstr = jnp.pad(wstr, ((0, 0), (0, 1)))[:, None, :]

    out = pl.pallas_call(
        _attn_body,
        grid=(H,),
        in_specs=[
            pl.BlockSpec((1, _TILE, _KWIN), lambda h: (h, 0, 0)),
            pl.BlockSpec((1, 1, 8), lambda h: (h, 0, 0)),
            pl.BlockSpec((1, N, HD), lambda h: (h, 0, 0)),
            pl.BlockSpec((1, N, HD), lambda h: (h, 0, 0)),
            pl.BlockSpec((1, N, HD), lambda h: (h, 0, 0)),
        ],
        out_specs=pl.BlockSpec((1, N, HD), lambda h: (h, 0, 0)),
        out_shape=jax.ShapeDtypeStruct((H, N, HD), jnp.float32),
    )(wband, wstr, q[0], k[0], v[0])
    return out[None]
